# baseline (speedup 1.0000x reference)
import functools

import jax
import jax.numpy as jnp
from jax import lax
from jax.experimental import pallas as pl
from jax.experimental.pallas import tpu as pltpu

try:
    from jax.experimental.pallas import tpu_sc as plsc
    _HAS_SC = True
except ImportError:
    plsc = None
    _HAS_SC = False

HIDDEN = 96
EMB = 32
K = 8192
N_TOK = 8192
T_TILE = 2048
K_CHUNK = 1024


def _dist_argmin_body(zf_ref, pw_ref, pb_ref, enorm_ref, embed_ref,
                      ze_out_ref, ind_out_ref):
    zf = zf_ref[...]
    pw = pw_ref[...]
    pb = pb_ref[...]
    ze = lax.dot_general(zf, pw, (((1,), (1,)), ((), ())),
                         preferred_element_type=jnp.float32) + pb
    ze_out_ref[...] = ze

    fnorm = jnp.sum(ze * ze, axis=1, keepdims=True)
    ze2 = ze + ze

    nchunks = K // K_CHUNK

    def _mm(c):
        e = embed_ref[pl.ds(c * K_CHUNK, K_CHUNK), :]
        return lax.dot_general(ze2, e, (((1,), (1,)), ((), ())),
                               preferred_element_type=jnp.float32)

    def _one_chunk(c, best_val, best_idxf):
        dot2 = _mm(c)
        en = enorm_ref[:, pl.ds(c * K_CHUNK, K_CHUNK)]
        dist = (fnorm - dot2) + en
        ngrp = K_CHUNK // 128
        pairs = [(dist[:, g * 128:(g + 1) * 128], jnp.float32(g))
                 for g in range(ngrp)]
        while len(pairs) > 1:
            nxt = []
            for (a, ga), (b, gb) in zip(pairs[0::2], pairs[1::2]):
                m = jnp.minimum(a, b)
                g = jnp.where(b < a, gb, ga)
                nxt.append((m, g))
            pairs = nxt
        m128, g128 = pairs[0]
        cmin = jnp.min(m128, axis=1, keepdims=True)
        lanef = lax.broadcasted_iota(jnp.int32, (1, 128), 1).astype(
            jnp.float32) + jnp.float32(c * K_CHUNK)
        colf = g128 * jnp.float32(128.0) + lanef
        cidxf = jnp.min(jnp.where(m128 == cmin, colf, jnp.float32(1e9)),
                        axis=1, keepdims=True)
        upd = cmin < best_val
        return (jnp.where(upd, cmin, best_val),
                jnp.where(upd, cidxf, best_idxf))

    UNROLL = 8

    def chunk_step(c2, carry):
        best_val, best_idxf = carry
        for u in range(UNROLL):
            best_val, best_idxf = _one_chunk(c2 * UNROLL + u,
                                             best_val, best_idxf)
        return (best_val, best_idxf)

    init = (jnp.full((T_TILE, 1), jnp.inf, jnp.float32),
            jnp.zeros((T_TILE, 1), jnp.float32))
    _, best_idxf = lax.fori_loop(0, nchunks // UNROLL, chunk_step, init)
    ind_out_ref[...] = best_idxf.astype(jnp.int32)


def _tc_dist_argmin(zf, proj_w, proj_b2, e_norms, embed):
    grid = (N_TOK // T_TILE,)
    return pl.pallas_call(
        _dist_argmin_body,
        grid=grid,
        in_specs=[
            pl.BlockSpec((T_TILE, HIDDEN), lambda i: (i, 0)),
            pl.BlockSpec((EMB, HIDDEN), lambda i: (0, 0)),
            pl.BlockSpec((1, EMB), lambda i: (0, 0)),
            pl.BlockSpec((1, K), lambda i: (0, 0)),
            pl.BlockSpec((K, EMB), lambda i: (0, 0)),
        ],
        out_specs=[
            pl.BlockSpec((T_TILE, EMB), lambda i: (i, 0)),
            pl.BlockSpec((T_TILE, 1), lambda i: (i, 0)),
        ],
        out_shape=[
            jax.ShapeDtypeStruct((N_TOK, EMB), jnp.float32),
            jax.ShapeDtypeStruct((N_TOK, 1), jnp.int32),
        ],
    )(zf, proj_w, proj_b2, e_norms, embed)


def _make_sc_gather():
    info = plsc.get_sparse_core_info()
    nc, ns = info.num_cores, info.num_subcores
    nw = nc * ns
    bw = N_TOK // nw
    mesh = plsc.VectorSubcoreMesh(core_axis_name="c", subcore_axis_name="s")

    @functools.partial(
        pl.kernel,
        mesh=mesh,
        out_type=[
            jax.ShapeDtypeStruct((N_TOK, EMB), jnp.float32),
            jax.ShapeDtypeStruct((nw, 16), jnp.float32),
        ],
        scratch_types=[
            pltpu.VMEM((bw,), jnp.int32),
            pltpu.VMEM((bw, 128), jnp.float32),
            pltpu.VMEM((bw, EMB), jnp.float32),
            pltpu.VMEM((bw, EMB), jnp.float32),
            pltpu.VMEM((16,), jnp.float32),
            pltpu.SemaphoreType.DMA,
        ],
    )
    def sc_kernel(embed_pad_hbm, idx_hbm, ze_hbm, zqst_hbm, part_hbm,
                  idx_v, rows_v, ze_v, out_v, acc_v, sem):
        wid = lax.axis_index("s") * nc + lax.axis_index("c")
        base = wid * bw
        pltpu.sync_copy(idx_hbm.at[pl.ds(base, bw)], idx_v)
        pltpu.async_copy(embed_pad_hbm.at[idx_v], rows_v, sem).wait()
        pltpu.sync_copy(ze_hbm.at[pl.ds(base, bw)], ze_v)

        def row_step(i, acc):
            for c in range(EMB // 16):
                zq = rows_v[i, pl.ds(c * 16, 16)]
                ze = ze_v[i, pl.ds(c * 16, 16)]
                d = zq - ze
                out_v[i, pl.ds(c * 16, 16)] = ze + d
                acc = acc + d * d
            return acc

        acc = lax.fori_loop(0, bw, row_step, jnp.zeros((16,), jnp.float32))
        acc_v[...] = acc
        pltpu.sync_copy(out_v, zqst_hbm.at[pl.ds(base, bw)])
        pltpu.sync_copy(acc_v, part_hbm.at[wid])

    return sc_kernel


def kernel(z, proj_w, proj_b, embed):
    B, C, H, W = z.shape
    zf = jnp.transpose(z, (0, 2, 3, 1)).reshape(N_TOK, C)
    pb2 = proj_b.reshape(1, EMB)
    e_norms = jnp.sum(embed ** 2, axis=1)[None, :]

    ze, ind = _tc_dist_argmin(zf, proj_w, pb2, e_norms, embed)
    idx_flat = ind.reshape(N_TOK)

    embed_pad = jnp.pad(embed, ((0, 0), (0, 128 - EMB)))
    sc = _make_sc_gather()
    zq_st, partials = sc(embed_pad, idx_flat, ze)

    diff = jnp.sum(partials) * jnp.float32(2.0 / (N_TOK * EMB))
    return (zq_st.reshape(B, H, W, EMB), diff,
            idx_flat.reshape(B, H, W))


# speedup vs baseline: 1.0589x; 1.0589x over previous
import functools

import jax
import jax.numpy as jnp
from jax import lax
from jax.experimental import pallas as pl
from jax.experimental.pallas import tpu as pltpu

try:
    from jax.experimental.pallas import tpu_sc as plsc
    _HAS_SC = True
except ImportError:
    plsc = None
    _HAS_SC = False

HIDDEN = 96
EMB = 32
K = 8192
N_TOK = 8192
T_TILE = 2048
K_CHUNK = 1024


def _dist_argmin_body(zf_ref, pw_ref, pb_ref, enorm_ref, embed_ref,
                      ze_out_ref, ind_out_ref):
    zf = zf_ref[...]
    pw = pw_ref[...]
    pb = pb_ref[...]
    ze = lax.dot_general(zf, pw, (((1,), (1,)), ((), ())),
                         preferred_element_type=jnp.float32) + pb
    ze_out_ref[...] = ze

    fnorm = jnp.sum(ze * ze, axis=1, keepdims=True)
    ze2 = ze + ze

    nchunks = K // K_CHUNK

    def _mm(c):
        e = embed_ref[pl.ds(c * K_CHUNK, K_CHUNK), :]
        return lax.dot_general(ze2, e, (((1,), (1,)), ((), ())),
                               preferred_element_type=jnp.float32)

    def _one_chunk(c, best_val, best_idxf):
        dot2 = _mm(c)
        en = enorm_ref[:, pl.ds(c * K_CHUNK, K_CHUNK)]
        dist = (fnorm - dot2) + en
        cmin = jnp.min(dist, axis=1, keepdims=True)
        colf = (lax.broadcasted_iota(jnp.int32, (1, K_CHUNK), 1)
                + c * K_CHUNK).astype(jnp.float32)
        cidxf = jnp.min(jnp.where(dist == cmin, colf, jnp.float32(1e9)),
                        axis=1, keepdims=True)
        upd = cmin < best_val
        return (jnp.where(upd, cmin, best_val),
                jnp.where(upd, cidxf, best_idxf))

    UNROLL = 8

    def chunk_step(c2, carry):
        best_val, best_idxf = carry
        for u in range(UNROLL):
            best_val, best_idxf = _one_chunk(c2 * UNROLL + u,
                                             best_val, best_idxf)
        return (best_val, best_idxf)

    init = (jnp.full((T_TILE, 1), jnp.inf, jnp.float32),
            jnp.zeros((T_TILE, 1), jnp.float32))
    _, best_idxf = lax.fori_loop(0, nchunks // UNROLL, chunk_step, init)
    ind_out_ref[...] = best_idxf.astype(jnp.int32)


def _tc_dist_argmin(zf, proj_w, proj_b2, e_norms, embed):
    grid = (N_TOK // T_TILE,)
    return pl.pallas_call(
        _dist_argmin_body,
        grid=grid,
        in_specs=[
            pl.BlockSpec((T_TILE, HIDDEN), lambda i: (i, 0)),
            pl.BlockSpec((EMB, HIDDEN), lambda i: (0, 0)),
            pl.BlockSpec((1, EMB), lambda i: (0, 0)),
            pl.BlockSpec((1, K), lambda i: (0, 0)),
            pl.BlockSpec((K, EMB), lambda i: (0, 0)),
        ],
        out_specs=[
            pl.BlockSpec((T_TILE, EMB), lambda i: (i, 0)),
            pl.BlockSpec((T_TILE, 1), lambda i: (i, 0)),
        ],
        out_shape=[
            jax.ShapeDtypeStruct((N_TOK, EMB), jnp.float32),
            jax.ShapeDtypeStruct((N_TOK, 1), jnp.int32),
        ],
    )(zf, proj_w, proj_b2, e_norms, embed)


def _make_sc_gather():
    info = plsc.get_sparse_core_info()
    nc, ns = info.num_cores, info.num_subcores
    nw = nc * ns
    bw = N_TOK // nw
    mesh = plsc.VectorSubcoreMesh(core_axis_name="c", subcore_axis_name="s")

    @functools.partial(
        pl.kernel,
        mesh=mesh,
        out_type=[
            jax.ShapeDtypeStruct((N_TOK, EMB), jnp.float32),
            jax.ShapeDtypeStruct((nw, 16), jnp.float32),
        ],
        scratch_types=[
            pltpu.VMEM((bw,), jnp.int32),
            pltpu.VMEM((bw, 128), jnp.float32),
            pltpu.VMEM((bw, EMB), jnp.float32),
            pltpu.VMEM((bw, EMB), jnp.float32),
            pltpu.VMEM((16,), jnp.float32),
            pltpu.SemaphoreType.DMA,
        ],
    )
    def sc_kernel(embed_pad_hbm, idx_hbm, ze_hbm, zqst_hbm, part_hbm,
                  idx_v, rows_v, ze_v, out_v, acc_v, sem):
        wid = lax.axis_index("s") * nc + lax.axis_index("c")
        base = wid * bw
        pltpu.sync_copy(idx_hbm.at[pl.ds(base, bw)], idx_v)
        pltpu.async_copy(embed_pad_hbm.at[idx_v], rows_v, sem).wait()
        pltpu.sync_copy(ze_hbm.at[pl.ds(base, bw)], ze_v)

        def row_step(i, acc):
            for c in range(EMB // 16):
                zq = rows_v[i, pl.ds(c * 16, 16)]
                ze = ze_v[i, pl.ds(c * 16, 16)]
                d = zq - ze
                out_v[i, pl.ds(c * 16, 16)] = ze + d
                acc = acc + d * d
            return acc

        acc = lax.fori_loop(0, bw, row_step, jnp.zeros((16,), jnp.float32))
        acc_v[...] = acc
        pltpu.sync_copy(out_v, zqst_hbm.at[pl.ds(base, bw)])
        pltpu.sync_copy(acc_v, part_hbm.at[wid])

    return sc_kernel


def kernel(z, proj_w, proj_b, embed):
    B, C, H, W = z.shape
    zf = jnp.transpose(z, (0, 2, 3, 1)).reshape(N_TOK, C)
    pb2 = proj_b.reshape(1, EMB)
    e_norms = jnp.sum(embed ** 2, axis=1)[None, :]

    ze, ind = _tc_dist_argmin(zf, proj_w, pb2, e_norms, embed)
    idx_flat = ind.reshape(N_TOK)

    embed_pad = jnp.pad(embed, ((0, 0), (0, 128 - EMB)))
    sc = _make_sc_gather()
    zq_st, partials = sc(embed_pad, idx_flat, ze)

    diff = jnp.sum(partials) * jnp.float32(2.0 / (N_TOK * EMB))
    return (zq_st.reshape(B, H, W, EMB), diff,
            idx_flat.reshape(B, H, W))
